# baseline (speedup 1.0000x reference)
import functools

import jax
import jax.numpy as jnp
from jax import lax
from jax.experimental import pallas as pl
from jax.experimental.pallas import tpu as pltpu
from jax.experimental.pallas import tpu_sc as plsc

_EPS = 1e-07
_NEG_RATIO = 3.0

_N = 16 * 512 * 512
_NC = 2
_NS = 16
_NW = _NC * _NS
_COLS = 512
_ROWS = _N // _COLS
_SC_ROWS = 4096
_ROWS_W = _SC_ROWS // _NW
_BAND = 16
_NCHUNKS = _ROWS_W // _BAND
_RING = 4
_LANES = 16
_TC_BLOCK = 256


def _sc_partials(pred, gt, mask):
    mesh = plsc.VectorSubcoreMesh(core_axis_name="c", subcore_axis_name="s")
    buf = pltpu.VMEM((_BAND, _COLS), jnp.float32)

    @functools.partial(
        pl.kernel,
        mesh=mesh,
        out_type=jax.ShapeDtypeStruct((_NW, 4 * _LANES), jnp.float32),
        scratch_types=[buf] * (3 * _RING)
        + [pltpu.SemaphoreType.DMA] * (3 * _RING)
        + [pltpu.VMEM((4 * _LANES,), jnp.float32)],
    )
    def body(p_hbm, g_hbm, m_hbm, out_hbm, *scr):
        wid = lax.axis_index("s") * _NC + lax.axis_index("c")
        base = wid * _ROWS_W
        hbm = (p_hbm, g_hbm, m_hbm)
        bufs = [scr[3 * i : 3 * i + 3] for i in range(_RING)]
        sems = [
            scr[3 * _RING + 3 * i : 3 * _RING + 3 * i + 3]
            for i in range(_RING)
        ]
        acc = scr[6 * _RING]
        last_row = base + (_NCHUNKS - 1) * _BAND

        def start(ci, slot):
            row0 = jnp.minimum(base + ci * _BAND, last_row)
            return [
                pltpu.async_copy(h.at[pl.ds(row0, _BAND)], b, s)
                for h, b, s in zip(hbm, bufs[slot], sems[slot])
            ]

        def wait(slot):
            for h, b, s in zip(hbm, bufs[slot], sems[slot]):
                pltpu.make_async_copy(h.at[pl.ds(0, _BAND)], b, s).wait()

        _SUB = 4

        def compute(slot, accs):
            pb, gb, mb = bufs[slot]

            def inner(r, accs2):
                accl = list(accs2)
                for cu in range(_COLS // _LANES):
                    c = cu * _LANES
                    k = cu % _SUB
                    p = pb[r, pl.ds(c, _LANES)]
                    g = gb[r, pl.ds(c, _LANES)]
                    m = mb[r, pl.ds(c, _LANES)]
                    pm = p * m
                    accl[k] = accl[k] + m
                    accl[_SUB + k] = accl[_SUB + k] + g * m
                    accl[2 * _SUB + k] = accl[2 * _SUB + k] + pm
                    accl[3 * _SUB + k] = accl[3 * _SUB + k] + pm * g
                return tuple(accl)

            return lax.fori_loop(0, _BAND, inner, accs, unroll=False)

        z = jnp.zeros((_LANES,), jnp.float32)
        for slot in range(_RING):
            start(slot, slot)

        def super_iter(t, accs):
            ci0 = t * _RING
            for b in range(_RING):
                wait(b)
                accs = compute(b, accs)
                start(ci0 + b + _RING, b)
            return accs

        accs = lax.fori_loop(
            0, _NCHUNKS // _RING, super_iter, (z,) * (4 * _SUB)
        )
        for slot in range(_RING):
            wait(slot)
        sums = [
            functools.reduce(
                lambda a, b: a + b, accs[i * _SUB : (i + 1) * _SUB]
            )
            for i in range(4)
        ]
        a_m, a_gm, a_pm, a_pgm = sums
        acc[pl.ds(0, _LANES)] = a_m
        acc[pl.ds(_LANES, _LANES)] = a_gm
        acc[pl.ds(2 * _LANES, _LANES)] = a_pm
        acc[pl.ds(3 * _LANES, _LANES)] = a_pgm
        pltpu.sync_copy(acc, out_hbm.at[wid])

    return body(pred, gt, mask)


def _tc_body(pa_ref, ga_ref, ma_ref, pb_ref, gb_ref, mb_ref, out_ref):
    i = pl.program_id(0)

    @pl.when(i == 0)
    def _init():
        out_ref[...] = jnp.zeros_like(out_ref)

    s_m = out_ref[0, :]
    s_gm = out_ref[1, :]
    s_pm = out_ref[2, :]
    s_pgm = out_ref[3, :]
    for p_ref, g_ref, m_ref in (
        (pa_ref, ga_ref, ma_ref),
        (pb_ref, gb_ref, mb_ref),
    ):
        p = p_ref[...]
        g = g_ref[...]
        m = m_ref[...]
        pm = p * m
        s_m = s_m + jnp.sum(m, axis=0)
        s_gm = s_gm + jnp.sum(g * m, axis=0)
        s_pm = s_pm + jnp.sum(pm, axis=0)
        s_pgm = s_pgm + jnp.sum(pm * g, axis=0)
    out_ref[0, :] = s_m
    out_ref[1, :] = s_gm
    out_ref[2, :] = s_pm
    out_ref[3, :] = s_pgm


def _tc_partials(p2, g2, m2):
    steps = (_ROWS - _SC_ROWS) // (2 * _TC_BLOCK)
    off = _SC_ROWS // _TC_BLOCK
    spec_a = pl.BlockSpec((_TC_BLOCK, _COLS), lambda i: (off + 2 * i, 0))
    spec_b = pl.BlockSpec((_TC_BLOCK, _COLS), lambda i: (off + 2 * i + 1, 0))
    return pl.pallas_call(
        _tc_body,
        grid=(steps,),
        in_specs=[spec_a, spec_a, spec_a, spec_b, spec_b, spec_b],
        out_specs=pl.BlockSpec((4, _COLS), lambda i: (0, 0)),
        out_shape=jax.ShapeDtypeStruct((4, _COLS), jnp.float32),
    )(p2, g2, m2, p2, g2, m2)


def _epilogue_body(part_ref, tc_ref, out_ref):
    x = part_ref[...]
    t = tc_ref[...]
    s_m = jnp.sum(x[:, 0:16]) + jnp.sum(t[0, :])
    s_gm = jnp.sum(x[:, 16:32]) + jnp.sum(t[1, :])
    s_pm = jnp.sum(x[:, 32:48]) + jnp.sum(t[2, :])
    s_pgm = jnp.sum(x[:, 48:64]) + jnp.sum(t[3, :])
    loss = 1.0 - 2.0 * s_pgm / (s_pm + s_gm + _EPS)
    pos = s_gm
    neg = jnp.minimum(s_m - s_gm, _NEG_RATIO * pos)
    balanced = loss * (pos + neg) / (pos + neg + _EPS)
    fallback = loss * pos / (pos + _EPS)
    out_ref[0, 0] = jnp.where(neg > 0.0, balanced, fallback)


@jax.jit
def kernel(pred, gt, mask):
    p2 = pred.reshape(_ROWS, _COLS)
    g2 = gt.reshape(_ROWS, _COLS)
    m2 = mask.reshape(_ROWS, _COLS)
    sc_part = _sc_partials(p2, g2, m2)
    tc_part = _tc_partials(p2, g2, m2)
    out = pl.pallas_call(
        _epilogue_body,
        out_specs=pl.BlockSpec((1, 1), memory_space=pltpu.SMEM),
        out_shape=jax.ShapeDtypeStruct((1, 1), jnp.float32),
    )(sc_part, tc_part)
    return out.reshape(())


# speedup vs baseline: 1.0650x; 1.0650x over previous
import functools

import jax
import jax.numpy as jnp
from jax import lax
from jax.experimental import pallas as pl
from jax.experimental.pallas import tpu as pltpu
from jax.experimental.pallas import tpu_sc as plsc

_EPS = 1e-07
_NEG_RATIO = 3.0

_N = 16 * 512 * 512
_NC = 2
_NS = 16
_NW = _NC * _NS
_COLS = 512
_ROWS = _N // _COLS
_SC_ROWS = 2048
_ROWS_W = _SC_ROWS // _NW
_BAND = 16
_NCHUNKS = _ROWS_W // _BAND
_RING = 4
_LANES = 16
_TC_BLOCK = 256


def _sc_partials(pred, gt, mask):
    mesh = plsc.VectorSubcoreMesh(core_axis_name="c", subcore_axis_name="s")
    buf = pltpu.VMEM((_BAND, _COLS), jnp.float32)

    @functools.partial(
        pl.kernel,
        mesh=mesh,
        out_type=jax.ShapeDtypeStruct((_NW, 4 * _LANES), jnp.float32),
        scratch_types=[buf] * (3 * _RING)
        + [pltpu.SemaphoreType.DMA] * (3 * _RING)
        + [pltpu.VMEM((4 * _LANES,), jnp.float32)],
    )
    def body(p_hbm, g_hbm, m_hbm, out_hbm, *scr):
        wid = lax.axis_index("s") * _NC + lax.axis_index("c")
        base = wid * _ROWS_W
        hbm = (p_hbm, g_hbm, m_hbm)
        bufs = [scr[3 * i : 3 * i + 3] for i in range(_RING)]
        sems = [
            scr[3 * _RING + 3 * i : 3 * _RING + 3 * i + 3]
            for i in range(_RING)
        ]
        acc = scr[6 * _RING]
        last_row = base + (_NCHUNKS - 1) * _BAND

        def start(ci, slot):
            row0 = jnp.minimum(base + ci * _BAND, last_row)
            return [
                pltpu.async_copy(h.at[pl.ds(row0, _BAND)], b, s)
                for h, b, s in zip(hbm, bufs[slot], sems[slot])
            ]

        def wait(slot):
            for h, b, s in zip(hbm, bufs[slot], sems[slot]):
                pltpu.make_async_copy(h.at[pl.ds(0, _BAND)], b, s).wait()

        _SUB = 4

        def compute(slot, accs):
            pb, gb, mb = bufs[slot]

            def inner(r, accs2):
                accl = list(accs2)
                for cu in range(_COLS // _LANES):
                    c = cu * _LANES
                    k = cu % _SUB
                    p = pb[r, pl.ds(c, _LANES)]
                    g = gb[r, pl.ds(c, _LANES)]
                    m = mb[r, pl.ds(c, _LANES)]
                    pm = p * m
                    accl[k] = accl[k] + m
                    accl[_SUB + k] = accl[_SUB + k] + g * m
                    accl[2 * _SUB + k] = accl[2 * _SUB + k] + pm
                    accl[3 * _SUB + k] = accl[3 * _SUB + k] + pm * g
                return tuple(accl)

            return lax.fori_loop(0, _BAND, inner, accs, unroll=False)

        z = jnp.zeros((_LANES,), jnp.float32)
        for slot in range(_RING):
            start(slot, slot)

        def super_iter(t, accs):
            ci0 = t * _RING
            for b in range(_RING):
                wait(b)
                accs = compute(b, accs)
                start(ci0 + b + _RING, b)
            return accs

        accs = lax.fori_loop(
            0, _NCHUNKS // _RING, super_iter, (z,) * (4 * _SUB)
        )
        for slot in range(_RING):
            wait(slot)
        sums = [
            functools.reduce(
                lambda a, b: a + b, accs[i * _SUB : (i + 1) * _SUB]
            )
            for i in range(4)
        ]
        a_m, a_gm, a_pm, a_pgm = sums
        acc[pl.ds(0, _LANES)] = a_m
        acc[pl.ds(_LANES, _LANES)] = a_gm
        acc[pl.ds(2 * _LANES, _LANES)] = a_pm
        acc[pl.ds(3 * _LANES, _LANES)] = a_pgm
        pltpu.sync_copy(acc, out_hbm.at[wid])

    return body(pred, gt, mask)


def _tc_body(pa_ref, ga_ref, ma_ref, pb_ref, gb_ref, mb_ref, out_ref):
    i = pl.program_id(0)

    @pl.when(i == 0)
    def _init():
        out_ref[...] = jnp.zeros_like(out_ref)

    s_m = out_ref[0, :]
    s_gm = out_ref[1, :]
    s_pm = out_ref[2, :]
    s_pgm = out_ref[3, :]
    for p_ref, g_ref, m_ref in (
        (pa_ref, ga_ref, ma_ref),
        (pb_ref, gb_ref, mb_ref),
    ):
        p = p_ref[...]
        g = g_ref[...]
        m = m_ref[...]
        pm = p * m
        s_m = s_m + jnp.sum(m, axis=0)
        s_gm = s_gm + jnp.sum(g * m, axis=0)
        s_pm = s_pm + jnp.sum(pm, axis=0)
        s_pgm = s_pgm + jnp.sum(pm * g, axis=0)
    out_ref[0, :] = s_m
    out_ref[1, :] = s_gm
    out_ref[2, :] = s_pm
    out_ref[3, :] = s_pgm


def _tc_partials(p2, g2, m2):
    steps = (_ROWS - _SC_ROWS) // (2 * _TC_BLOCK)
    off = _SC_ROWS // _TC_BLOCK
    spec_a = pl.BlockSpec((_TC_BLOCK, _COLS), lambda i: (off + 2 * i, 0))
    spec_b = pl.BlockSpec((_TC_BLOCK, _COLS), lambda i: (off + 2 * i + 1, 0))
    return pl.pallas_call(
        _tc_body,
        grid=(steps,),
        in_specs=[spec_a, spec_a, spec_a, spec_b, spec_b, spec_b],
        out_specs=pl.BlockSpec((4, _COLS), lambda i: (0, 0)),
        out_shape=jax.ShapeDtypeStruct((4, _COLS), jnp.float32),
    )(p2, g2, m2, p2, g2, m2)


def _epilogue_body(part_ref, tc_ref, out_ref):
    x = part_ref[...]
    t = tc_ref[...]
    s_m = jnp.sum(x[:, 0:16]) + jnp.sum(t[0, :])
    s_gm = jnp.sum(x[:, 16:32]) + jnp.sum(t[1, :])
    s_pm = jnp.sum(x[:, 32:48]) + jnp.sum(t[2, :])
    s_pgm = jnp.sum(x[:, 48:64]) + jnp.sum(t[3, :])
    loss = 1.0 - 2.0 * s_pgm / (s_pm + s_gm + _EPS)
    pos = s_gm
    neg = jnp.minimum(s_m - s_gm, _NEG_RATIO * pos)
    balanced = loss * (pos + neg) / (pos + neg + _EPS)
    fallback = loss * pos / (pos + _EPS)
    out_ref[0, 0] = jnp.where(neg > 0.0, balanced, fallback)


@jax.jit
def kernel(pred, gt, mask):
    p2 = pred.reshape(_ROWS, _COLS)
    g2 = gt.reshape(_ROWS, _COLS)
    m2 = mask.reshape(_ROWS, _COLS)
    sc_part = _sc_partials(p2, g2, m2)
    tc_part = _tc_partials(p2, g2, m2)
    out = pl.pallas_call(
        _epilogue_body,
        out_specs=pl.BlockSpec((1, 1), memory_space=pltpu.SMEM),
        out_shape=jax.ShapeDtypeStruct((1, 1), jnp.float32),
    )(sc_part, tc_part)
    return out.reshape(())


# speedup vs baseline: 1.0709x; 1.0055x over previous
import functools

import jax
import jax.numpy as jnp
from jax import lax
from jax.experimental import pallas as pl
from jax.experimental.pallas import tpu as pltpu
from jax.experimental.pallas import tpu_sc as plsc

_EPS = 1e-07
_NEG_RATIO = 3.0

_N = 16 * 512 * 512
_NC = 2
_NS = 16
_NW = _NC * _NS
_COLS = 512
_ROWS = _N // _COLS
_SC_ROWS = 2048
_ROWS_W = _SC_ROWS // _NW
_BAND = 16
_NCHUNKS = _ROWS_W // _BAND
_RING = 4
_LANES = 16
_TC_BLOCK = 128


def _sc_partials(pred, gt, mask):
    mesh = plsc.VectorSubcoreMesh(core_axis_name="c", subcore_axis_name="s")
    buf = pltpu.VMEM((_BAND, _COLS), jnp.float32)

    @functools.partial(
        pl.kernel,
        mesh=mesh,
        out_type=jax.ShapeDtypeStruct((_NW, 4 * _LANES), jnp.float32),
        scratch_types=[buf] * (3 * _RING)
        + [pltpu.SemaphoreType.DMA] * (3 * _RING)
        + [pltpu.VMEM((4 * _LANES,), jnp.float32)],
    )
    def body(p_hbm, g_hbm, m_hbm, out_hbm, *scr):
        wid = lax.axis_index("s") * _NC + lax.axis_index("c")
        base = wid * _ROWS_W
        hbm = (p_hbm, g_hbm, m_hbm)
        bufs = [scr[3 * i : 3 * i + 3] for i in range(_RING)]
        sems = [
            scr[3 * _RING + 3 * i : 3 * _RING + 3 * i + 3]
            for i in range(_RING)
        ]
        acc = scr[6 * _RING]
        last_row = base + (_NCHUNKS - 1) * _BAND

        def start(ci, slot):
            row0 = jnp.minimum(base + ci * _BAND, last_row)
            return [
                pltpu.async_copy(h.at[pl.ds(row0, _BAND)], b, s)
                for h, b, s in zip(hbm, bufs[slot], sems[slot])
            ]

        def wait(slot):
            for h, b, s in zip(hbm, bufs[slot], sems[slot]):
                pltpu.make_async_copy(h.at[pl.ds(0, _BAND)], b, s).wait()

        _SUB = 4

        def compute(slot, accs):
            pb, gb, mb = bufs[slot]

            def inner(r, accs2):
                accl = list(accs2)
                for cu in range(_COLS // _LANES):
                    c = cu * _LANES
                    k = cu % _SUB
                    p = pb[r, pl.ds(c, _LANES)]
                    g = gb[r, pl.ds(c, _LANES)]
                    m = mb[r, pl.ds(c, _LANES)]
                    pm = p * m
                    accl[k] = accl[k] + m
                    accl[_SUB + k] = accl[_SUB + k] + g * m
                    accl[2 * _SUB + k] = accl[2 * _SUB + k] + pm
                    accl[3 * _SUB + k] = accl[3 * _SUB + k] + pm * g
                return tuple(accl)

            return lax.fori_loop(0, _BAND, inner, accs, unroll=False)

        z = jnp.zeros((_LANES,), jnp.float32)
        for slot in range(_RING):
            start(slot, slot)

        def super_iter(t, accs):
            ci0 = t * _RING
            for b in range(_RING):
                wait(b)
                accs = compute(b, accs)
                start(ci0 + b + _RING, b)
            return accs

        accs = lax.fori_loop(
            0, _NCHUNKS // _RING, super_iter, (z,) * (4 * _SUB)
        )
        for slot in range(_RING):
            wait(slot)
        sums = [
            functools.reduce(
                lambda a, b: a + b, accs[i * _SUB : (i + 1) * _SUB]
            )
            for i in range(4)
        ]
        a_m, a_gm, a_pm, a_pgm = sums
        acc[pl.ds(0, _LANES)] = a_m
        acc[pl.ds(_LANES, _LANES)] = a_gm
        acc[pl.ds(2 * _LANES, _LANES)] = a_pm
        acc[pl.ds(3 * _LANES, _LANES)] = a_pgm
        pltpu.sync_copy(acc, out_hbm.at[wid])

    return body(pred, gt, mask)


_TC_WAYS = 4


def _tc_body(*refs):
    out_ref = refs[-1]
    i = pl.program_id(0)

    @pl.when(i == 0)
    def _init():
        out_ref[...] = jnp.zeros_like(out_ref)

    s_m = out_ref[0, :]
    s_gm = out_ref[1, :]
    s_pm = out_ref[2, :]
    s_pgm = out_ref[3, :]
    for w in range(_TC_WAYS):
        p = refs[3 * w][...]
        g = refs[3 * w + 1][...]
        m = refs[3 * w + 2][...]
        pm = p * m
        s_m = s_m + jnp.sum(m, axis=0)
        s_gm = s_gm + jnp.sum(g * m, axis=0)
        s_pm = s_pm + jnp.sum(pm, axis=0)
        s_pgm = s_pgm + jnp.sum(pm * g, axis=0)
    out_ref[0, :] = s_m
    out_ref[1, :] = s_gm
    out_ref[2, :] = s_pm
    out_ref[3, :] = s_pgm


def _tc_partials(p2, g2, m2):
    steps = (_ROWS - _SC_ROWS) // (_TC_WAYS * _TC_BLOCK)
    off = _SC_ROWS // _TC_BLOCK

    def mk_spec(k):
        return pl.BlockSpec(
            (_TC_BLOCK, _COLS), lambda i: (off + _TC_WAYS * i + k, 0)
        )

    in_specs = []
    operands = []
    for k in range(_TC_WAYS):
        spec = mk_spec(k)
        in_specs += [spec, spec, spec]
        operands += [p2, g2, m2]
    return pl.pallas_call(
        _tc_body,
        grid=(steps,),
        in_specs=in_specs,
        out_specs=pl.BlockSpec((4, _COLS), lambda i: (0, 0)),
        out_shape=jax.ShapeDtypeStruct((4, _COLS), jnp.float32),
    )(*operands)


def _epilogue_body(part_ref, tc_ref, out_ref):
    x = part_ref[...]
    t = tc_ref[...]
    s_m = jnp.sum(x[:, 0:16]) + jnp.sum(t[0, :])
    s_gm = jnp.sum(x[:, 16:32]) + jnp.sum(t[1, :])
    s_pm = jnp.sum(x[:, 32:48]) + jnp.sum(t[2, :])
    s_pgm = jnp.sum(x[:, 48:64]) + jnp.sum(t[3, :])
    loss = 1.0 - 2.0 * s_pgm / (s_pm + s_gm + _EPS)
    pos = s_gm
    neg = jnp.minimum(s_m - s_gm, _NEG_RATIO * pos)
    balanced = loss * (pos + neg) / (pos + neg + _EPS)
    fallback = loss * pos / (pos + _EPS)
    out_ref[0, 0] = jnp.where(neg > 0.0, balanced, fallback)


@jax.jit
def kernel(pred, gt, mask):
    p2 = pred.reshape(_ROWS, _COLS)
    g2 = gt.reshape(_ROWS, _COLS)
    m2 = mask.reshape(_ROWS, _COLS)
    sc_part = _sc_partials(p2, g2, m2)
    tc_part = _tc_partials(p2, g2, m2)
    out = pl.pallas_call(
        _epilogue_body,
        out_specs=pl.BlockSpec((1, 1), memory_space=pltpu.SMEM),
        out_shape=jax.ShapeDtypeStruct((1, 1), jnp.float32),
    )(sc_part, tc_part)
    return out.reshape(())


# speedup vs baseline: 1.0982x; 1.0255x over previous
import functools

import jax
import jax.numpy as jnp
from jax import lax
from jax.experimental import pallas as pl
from jax.experimental.pallas import tpu as pltpu
from jax.experimental.pallas import tpu_sc as plsc

_EPS = 1e-07
_NEG_RATIO = 3.0

_N = 16 * 512 * 512
_NC = 2
_NS = 16
_NW = _NC * _NS
_COLS = 512
_ROWS = _N // _COLS
_SC_ROWS = 3072
_ROWS_W = _SC_ROWS // _NW
_BAND = 16
_NCHUNKS = _ROWS_W // _BAND
_RING = 2
_LANES = 16
_TC_BLOCK = 128


def _sc_partials(pred, gt, mask):
    mesh = plsc.VectorSubcoreMesh(core_axis_name="c", subcore_axis_name="s")
    buf = pltpu.VMEM((_BAND, _COLS), jnp.float32)

    @functools.partial(
        pl.kernel,
        mesh=mesh,
        out_type=jax.ShapeDtypeStruct((_NW, 4 * _LANES), jnp.float32),
        scratch_types=[buf] * (3 * _RING)
        + [pltpu.SemaphoreType.DMA] * (3 * _RING)
        + [pltpu.VMEM((4 * _LANES,), jnp.float32)],
    )
    def body(p_hbm, g_hbm, m_hbm, out_hbm, *scr):
        wid = lax.axis_index("s") * _NC + lax.axis_index("c")
        base = wid * _ROWS_W
        hbm = (p_hbm, g_hbm, m_hbm)
        bufs = [scr[3 * i : 3 * i + 3] for i in range(_RING)]
        sems = [
            scr[3 * _RING + 3 * i : 3 * _RING + 3 * i + 3]
            for i in range(_RING)
        ]
        acc = scr[6 * _RING]
        last_row = base + (_NCHUNKS - 1) * _BAND

        def start(ci, slot):
            row0 = jnp.minimum(base + ci * _BAND, last_row)
            return [
                pltpu.async_copy(h.at[pl.ds(row0, _BAND)], b, s)
                for h, b, s in zip(hbm, bufs[slot], sems[slot])
            ]

        def wait(slot):
            for h, b, s in zip(hbm, bufs[slot], sems[slot]):
                pltpu.make_async_copy(h.at[pl.ds(0, _BAND)], b, s).wait()

        _SUB = 4

        def compute(slot, accs):
            pb, gb, mb = bufs[slot]

            def inner(r, accs2):
                accl = list(accs2)
                for cu in range(_COLS // _LANES):
                    c = cu * _LANES
                    k = cu % _SUB
                    p = pb[r, pl.ds(c, _LANES)]
                    g = gb[r, pl.ds(c, _LANES)]
                    m = mb[r, pl.ds(c, _LANES)]
                    pm = p * m
                    accl[k] = accl[k] + m
                    accl[_SUB + k] = accl[_SUB + k] + g * m
                    accl[2 * _SUB + k] = accl[2 * _SUB + k] + pm
                    accl[3 * _SUB + k] = accl[3 * _SUB + k] + pm * g
                return tuple(accl)

            return lax.fori_loop(0, _BAND, inner, accs, unroll=False)

        z = jnp.zeros((_LANES,), jnp.float32)
        for slot in range(_RING):
            start(slot, slot)

        def super_iter(t, accs):
            ci0 = t * _RING
            for b in range(_RING):
                wait(b)
                accs = compute(b, accs)
                start(ci0 + b + _RING, b)
            return accs

        accs = lax.fori_loop(
            0, _NCHUNKS // _RING, super_iter, (z,) * (4 * _SUB)
        )
        for slot in range(_RING):
            wait(slot)
        sums = [
            functools.reduce(
                lambda a, b: a + b, accs[i * _SUB : (i + 1) * _SUB]
            )
            for i in range(4)
        ]
        a_m, a_gm, a_pm, a_pgm = sums
        acc[pl.ds(0, _LANES)] = a_m
        acc[pl.ds(_LANES, _LANES)] = a_gm
        acc[pl.ds(2 * _LANES, _LANES)] = a_pm
        acc[pl.ds(3 * _LANES, _LANES)] = a_pgm
        pltpu.sync_copy(acc, out_hbm.at[wid])

    return body(pred, gt, mask)


_TC_WAYS = 4


def _tc_body(*refs):
    out_ref = refs[-1]
    i = pl.program_id(0)

    @pl.when(i == 0)
    def _init():
        out_ref[...] = jnp.zeros_like(out_ref)

    s_m = out_ref[0, :]
    s_gm = out_ref[1, :]
    s_pm = out_ref[2, :]
    s_pgm = out_ref[3, :]
    for w in range(_TC_WAYS):
        p = refs[3 * w][...]
        g = refs[3 * w + 1][...]
        m = refs[3 * w + 2][...]
        pm = p * m
        s_m = s_m + jnp.sum(m, axis=0)
        s_gm = s_gm + jnp.sum(g * m, axis=0)
        s_pm = s_pm + jnp.sum(pm, axis=0)
        s_pgm = s_pgm + jnp.sum(pm * g, axis=0)
    out_ref[0, :] = s_m
    out_ref[1, :] = s_gm
    out_ref[2, :] = s_pm
    out_ref[3, :] = s_pgm


def _tc_partials(p2, g2, m2):
    steps = (_ROWS - _SC_ROWS) // (_TC_WAYS * _TC_BLOCK)
    off = _SC_ROWS // _TC_BLOCK

    def mk_spec(k):
        return pl.BlockSpec(
            (_TC_BLOCK, _COLS), lambda i: (off + _TC_WAYS * i + k, 0)
        )

    in_specs = []
    operands = []
    for k in range(_TC_WAYS):
        spec = mk_spec(k)
        in_specs += [spec, spec, spec]
        operands += [p2, g2, m2]
    return pl.pallas_call(
        _tc_body,
        grid=(steps,),
        in_specs=in_specs,
        out_specs=pl.BlockSpec((4, _COLS), lambda i: (0, 0)),
        out_shape=jax.ShapeDtypeStruct((4, _COLS), jnp.float32),
    )(*operands)


def _epilogue_body(part_ref, tc_ref, out_ref):
    x = part_ref[...]
    t = tc_ref[...]
    s_m = jnp.sum(x[:, 0:16]) + jnp.sum(t[0, :])
    s_gm = jnp.sum(x[:, 16:32]) + jnp.sum(t[1, :])
    s_pm = jnp.sum(x[:, 32:48]) + jnp.sum(t[2, :])
    s_pgm = jnp.sum(x[:, 48:64]) + jnp.sum(t[3, :])
    loss = 1.0 - 2.0 * s_pgm / (s_pm + s_gm + _EPS)
    pos = s_gm
    neg = jnp.minimum(s_m - s_gm, _NEG_RATIO * pos)
    balanced = loss * (pos + neg) / (pos + neg + _EPS)
    fallback = loss * pos / (pos + _EPS)
    out_ref[0, 0] = jnp.where(neg > 0.0, balanced, fallback)


@jax.jit
def kernel(pred, gt, mask):
    p2 = pred.reshape(_ROWS, _COLS)
    g2 = gt.reshape(_ROWS, _COLS)
    m2 = mask.reshape(_ROWS, _COLS)
    sc_part = _sc_partials(p2, g2, m2)
    tc_part = _tc_partials(p2, g2, m2)
    out = pl.pallas_call(
        _epilogue_body,
        out_specs=pl.BlockSpec((1, 1), memory_space=pltpu.SMEM),
        out_shape=jax.ShapeDtypeStruct((1, 1), jnp.float32),
    )(sc_part, tc_part)
    return out.reshape(())


# speedup vs baseline: 1.1058x; 1.0070x over previous
import functools

import jax
import jax.numpy as jnp
from jax import lax
from jax.experimental import pallas as pl
from jax.experimental.pallas import tpu as pltpu
from jax.experimental.pallas import tpu_sc as plsc

_EPS = 1e-07
_NEG_RATIO = 3.0

_N = 16 * 512 * 512
_NC = 2
_NS = 16
_NW = _NC * _NS
_COLS = 512
_ROWS = _N // _COLS
_SC_ROWS = 2560
_ROWS_W = _SC_ROWS // _NW
_BAND = 16
_NCHUNKS = _ROWS_W // _BAND
_RING = 5
_LANES = 16
_TC_BLOCK = 128


def _sc_partials(pred, gt, mask):
    mesh = plsc.VectorSubcoreMesh(core_axis_name="c", subcore_axis_name="s")
    buf = pltpu.VMEM((_BAND, _COLS), jnp.float32)

    @functools.partial(
        pl.kernel,
        mesh=mesh,
        out_type=jax.ShapeDtypeStruct((_NW, 4 * _LANES), jnp.float32),
        scratch_types=[buf] * (3 * _RING)
        + [pltpu.SemaphoreType.DMA] * (3 * _RING)
        + [pltpu.VMEM((4 * _LANES,), jnp.float32)],
    )
    def body(p_hbm, g_hbm, m_hbm, out_hbm, *scr):
        wid = lax.axis_index("s") * _NC + lax.axis_index("c")
        base = wid * _ROWS_W
        hbm = (p_hbm, g_hbm, m_hbm)
        bufs = [scr[3 * i : 3 * i + 3] for i in range(_RING)]
        sems = [
            scr[3 * _RING + 3 * i : 3 * _RING + 3 * i + 3]
            for i in range(_RING)
        ]
        acc = scr[6 * _RING]
        last_row = base + (_NCHUNKS - 1) * _BAND

        def start(ci, slot):
            row0 = jnp.minimum(base + ci * _BAND, last_row)
            return [
                pltpu.async_copy(h.at[pl.ds(row0, _BAND)], b, s)
                for h, b, s in zip(hbm, bufs[slot], sems[slot])
            ]

        def wait(slot):
            for h, b, s in zip(hbm, bufs[slot], sems[slot]):
                pltpu.make_async_copy(h.at[pl.ds(0, _BAND)], b, s).wait()

        _SUB = 4

        def compute(slot, accs):
            pb, gb, mb = bufs[slot]

            def inner(r, accs2):
                accl = list(accs2)
                for cu in range(_COLS // _LANES):
                    c = cu * _LANES
                    k = cu % _SUB
                    p = pb[r, pl.ds(c, _LANES)]
                    g = gb[r, pl.ds(c, _LANES)]
                    m = mb[r, pl.ds(c, _LANES)]
                    pm = p * m
                    accl[k] = accl[k] + m
                    accl[_SUB + k] = accl[_SUB + k] + g * m
                    accl[2 * _SUB + k] = accl[2 * _SUB + k] + pm
                    accl[3 * _SUB + k] = accl[3 * _SUB + k] + pm * g
                return tuple(accl)

            return lax.fori_loop(0, _BAND, inner, accs, unroll=False)

        z = jnp.zeros((_LANES,), jnp.float32)
        for slot in range(_RING):
            start(slot, slot)

        accs = (z,) * (4 * _SUB)
        if _NCHUNKS == _RING:
            for b in range(_RING):
                wait(b)
                accs = compute(b, accs)
        else:

            def super_iter(t, accs):
                ci0 = t * _RING
                for b in range(_RING):
                    wait(b)
                    accs = compute(b, accs)
                    start(ci0 + b + _RING, b)
                return accs

            accs = lax.fori_loop(0, _NCHUNKS // _RING, super_iter, accs)
            for slot in range(_RING):
                wait(slot)
        sums = [
            functools.reduce(
                lambda a, b: a + b, accs[i * _SUB : (i + 1) * _SUB]
            )
            for i in range(4)
        ]
        a_m, a_gm, a_pm, a_pgm = sums
        acc[pl.ds(0, _LANES)] = a_m
        acc[pl.ds(_LANES, _LANES)] = a_gm
        acc[pl.ds(2 * _LANES, _LANES)] = a_pm
        acc[pl.ds(3 * _LANES, _LANES)] = a_pgm
        pltpu.sync_copy(acc, out_hbm.at[wid])

    return body(pred, gt, mask)


_TC_WAYS = 4


def _tc_body(*refs):
    out_ref = refs[-1]
    i = pl.program_id(0)

    @pl.when(i == 0)
    def _init():
        out_ref[...] = jnp.zeros_like(out_ref)

    s_m = out_ref[0, :]
    s_gm = out_ref[1, :]
    s_pm = out_ref[2, :]
    s_pgm = out_ref[3, :]
    for w in range(_TC_WAYS):
        p = refs[3 * w][...]
        g = refs[3 * w + 1][...]
        m = refs[3 * w + 2][...]
        pm = p * m
        s_m = s_m + jnp.sum(m, axis=0)
        s_gm = s_gm + jnp.sum(g * m, axis=0)
        s_pm = s_pm + jnp.sum(pm, axis=0)
        s_pgm = s_pgm + jnp.sum(pm * g, axis=0)
    out_ref[0, :] = s_m
    out_ref[1, :] = s_gm
    out_ref[2, :] = s_pm
    out_ref[3, :] = s_pgm


def _tc_partials(p2, g2, m2):
    steps = (_ROWS - _SC_ROWS) // (_TC_WAYS * _TC_BLOCK)
    off = _SC_ROWS // _TC_BLOCK

    def mk_spec(k):
        return pl.BlockSpec(
            (_TC_BLOCK, _COLS), lambda i: (off + _TC_WAYS * i + k, 0)
        )

    in_specs = []
    operands = []
    for k in range(_TC_WAYS):
        spec = mk_spec(k)
        in_specs += [spec, spec, spec]
        operands += [p2, g2, m2]
    return pl.pallas_call(
        _tc_body,
        grid=(steps,),
        in_specs=in_specs,
        out_specs=pl.BlockSpec((4, _COLS), lambda i: (0, 0)),
        out_shape=jax.ShapeDtypeStruct((4, _COLS), jnp.float32),
    )(*operands)


def _epilogue_body(part_ref, tc_ref, out_ref):
    x = part_ref[...]
    t = tc_ref[...]
    s_m = jnp.sum(x[:, 0:16]) + jnp.sum(t[0, :])
    s_gm = jnp.sum(x[:, 16:32]) + jnp.sum(t[1, :])
    s_pm = jnp.sum(x[:, 32:48]) + jnp.sum(t[2, :])
    s_pgm = jnp.sum(x[:, 48:64]) + jnp.sum(t[3, :])
    loss = 1.0 - 2.0 * s_pgm / (s_pm + s_gm + _EPS)
    pos = s_gm
    neg = jnp.minimum(s_m - s_gm, _NEG_RATIO * pos)
    balanced = loss * (pos + neg) / (pos + neg + _EPS)
    fallback = loss * pos / (pos + _EPS)
    out_ref[0, 0] = jnp.where(neg > 0.0, balanced, fallback)


@jax.jit
def kernel(pred, gt, mask):
    p2 = pred.reshape(_ROWS, _COLS)
    g2 = gt.reshape(_ROWS, _COLS)
    m2 = mask.reshape(_ROWS, _COLS)
    sc_part = _sc_partials(p2, g2, m2)
    tc_part = _tc_partials(p2, g2, m2)
    out = pl.pallas_call(
        _epilogue_body,
        out_specs=pl.BlockSpec((1, 1), memory_space=pltpu.SMEM),
        out_shape=jax.ShapeDtypeStruct((1, 1), jnp.float32),
    )(sc_part, tc_part)
    return out.reshape(())


# speedup vs baseline: 1.1162x; 1.0094x over previous
import functools

import jax
import jax.numpy as jnp
from jax import lax
from jax.experimental import pallas as pl
from jax.experimental.pallas import tpu as pltpu
from jax.experimental.pallas import tpu_sc as plsc

_EPS = 1e-07
_NEG_RATIO = 3.0

_N = 16 * 512 * 512
_NC = 2
_NS = 16
_NW = _NC * _NS
_COLS = 512
_ROWS = _N // _COLS
_SC_ROWS = 2560
_ROWS_W = _SC_ROWS // _NW
_BAND = 16
_NCHUNKS = _ROWS_W // _BAND
_RING = 5
_LANES = 16
_TC_BLOCK = 64


def _sc_partials(pred, gt, mask):
    mesh = plsc.VectorSubcoreMesh(core_axis_name="c", subcore_axis_name="s")
    buf = pltpu.VMEM((_BAND, _COLS), jnp.float32)

    @functools.partial(
        pl.kernel,
        mesh=mesh,
        out_type=jax.ShapeDtypeStruct((_NW, 4 * _LANES), jnp.float32),
        scratch_types=[buf] * (3 * _RING)
        + [pltpu.SemaphoreType.DMA] * (3 * _RING)
        + [pltpu.VMEM((4 * _LANES,), jnp.float32)],
    )
    def body(p_hbm, g_hbm, m_hbm, out_hbm, *scr):
        wid = lax.axis_index("s") * _NC + lax.axis_index("c")
        base = wid * _ROWS_W
        hbm = (p_hbm, g_hbm, m_hbm)
        bufs = [scr[3 * i : 3 * i + 3] for i in range(_RING)]
        sems = [
            scr[3 * _RING + 3 * i : 3 * _RING + 3 * i + 3]
            for i in range(_RING)
        ]
        acc = scr[6 * _RING]
        last_row = base + (_NCHUNKS - 1) * _BAND

        def start(ci, slot):
            row0 = jnp.minimum(base + ci * _BAND, last_row)
            return [
                pltpu.async_copy(h.at[pl.ds(row0, _BAND)], b, s)
                for h, b, s in zip(hbm, bufs[slot], sems[slot])
            ]

        def wait(slot):
            for h, b, s in zip(hbm, bufs[slot], sems[slot]):
                pltpu.make_async_copy(h.at[pl.ds(0, _BAND)], b, s).wait()

        _SUB = 4

        def compute(slot, accs):
            pb, gb, mb = bufs[slot]

            def inner(r, accs2):
                accl = list(accs2)
                for cu in range(_COLS // _LANES):
                    c = cu * _LANES
                    k = cu % _SUB
                    p = pb[r, pl.ds(c, _LANES)]
                    g = gb[r, pl.ds(c, _LANES)]
                    m = mb[r, pl.ds(c, _LANES)]
                    pm = p * m
                    accl[k] = accl[k] + m
                    accl[_SUB + k] = accl[_SUB + k] + g * m
                    accl[2 * _SUB + k] = accl[2 * _SUB + k] + pm
                    accl[3 * _SUB + k] = accl[3 * _SUB + k] + pm * g
                return tuple(accl)

            return lax.fori_loop(0, _BAND, inner, accs, unroll=False)

        z = jnp.zeros((_LANES,), jnp.float32)
        for slot in range(_RING):
            start(slot, slot)

        accs = (z,) * (4 * _SUB)
        if _NCHUNKS == _RING:
            for b in range(_RING):
                wait(b)
                accs = compute(b, accs)
        else:

            def super_iter(t, accs):
                ci0 = t * _RING
                for b in range(_RING):
                    wait(b)
                    accs = compute(b, accs)
                    start(ci0 + b + _RING, b)
                return accs

            accs = lax.fori_loop(0, _NCHUNKS // _RING, super_iter, accs)
            for slot in range(_RING):
                wait(slot)
        sums = [
            functools.reduce(
                lambda a, b: a + b, accs[i * _SUB : (i + 1) * _SUB]
            )
            for i in range(4)
        ]
        a_m, a_gm, a_pm, a_pgm = sums
        acc[pl.ds(0, _LANES)] = a_m
        acc[pl.ds(_LANES, _LANES)] = a_gm
        acc[pl.ds(2 * _LANES, _LANES)] = a_pm
        acc[pl.ds(3 * _LANES, _LANES)] = a_pgm
        pltpu.sync_copy(acc, out_hbm.at[wid])

    return body(pred, gt, mask)


_TC_WAYS = 8


def _tc_body(*refs):
    out_ref = refs[-1]
    i = pl.program_id(0)

    @pl.when(i == 0)
    def _init():
        out_ref[...] = jnp.zeros_like(out_ref)

    s_m = out_ref[0, :]
    s_gm = out_ref[1, :]
    s_pm = out_ref[2, :]
    s_pgm = out_ref[3, :]
    for w in range(_TC_WAYS):
        p = refs[3 * w][...]
        g = refs[3 * w + 1][...]
        m = refs[3 * w + 2][...]
        pm = p * m
        s_m = s_m + jnp.sum(m, axis=0)
        s_gm = s_gm + jnp.sum(g * m, axis=0)
        s_pm = s_pm + jnp.sum(pm, axis=0)
        s_pgm = s_pgm + jnp.sum(pm * g, axis=0)
    out_ref[0, :] = s_m
    out_ref[1, :] = s_gm
    out_ref[2, :] = s_pm
    out_ref[3, :] = s_pgm


def _tc_partials(p2, g2, m2):
    steps = (_ROWS - _SC_ROWS) // (_TC_WAYS * _TC_BLOCK)
    off = _SC_ROWS // _TC_BLOCK

    def mk_spec(k):
        return pl.BlockSpec(
            (_TC_BLOCK, _COLS), lambda i: (off + _TC_WAYS * i + k, 0)
        )

    in_specs = []
    operands = []
    for k in range(_TC_WAYS):
        spec = mk_spec(k)
        in_specs += [spec, spec, spec]
        operands += [p2, g2, m2]
    return pl.pallas_call(
        _tc_body,
        grid=(steps,),
        in_specs=in_specs,
        out_specs=pl.BlockSpec((4, _COLS), lambda i: (0, 0)),
        out_shape=jax.ShapeDtypeStruct((4, _COLS), jnp.float32),
    )(*operands)


def _epilogue_body(part_ref, tc_ref, out_ref):
    x = part_ref[...]
    t = tc_ref[...]
    s_m = jnp.sum(x[:, 0:16]) + jnp.sum(t[0, :])
    s_gm = jnp.sum(x[:, 16:32]) + jnp.sum(t[1, :])
    s_pm = jnp.sum(x[:, 32:48]) + jnp.sum(t[2, :])
    s_pgm = jnp.sum(x[:, 48:64]) + jnp.sum(t[3, :])
    loss = 1.0 - 2.0 * s_pgm / (s_pm + s_gm + _EPS)
    pos = s_gm
    neg = jnp.minimum(s_m - s_gm, _NEG_RATIO * pos)
    balanced = loss * (pos + neg) / (pos + neg + _EPS)
    fallback = loss * pos / (pos + _EPS)
    out_ref[0, 0] = jnp.where(neg > 0.0, balanced, fallback)


@jax.jit
def kernel(pred, gt, mask):
    p2 = pred.reshape(_ROWS, _COLS)
    g2 = gt.reshape(_ROWS, _COLS)
    m2 = mask.reshape(_ROWS, _COLS)
    sc_part = _sc_partials(p2, g2, m2)
    tc_part = _tc_partials(p2, g2, m2)
    out = pl.pallas_call(
        _epilogue_body,
        out_specs=pl.BlockSpec((1, 1), memory_space=pltpu.SMEM),
        out_shape=jax.ShapeDtypeStruct((1, 1), jnp.float32),
    )(sc_part, tc_part)
    return out.reshape(())
